# baseline (speedup 1.0000x reference)
import functools

import jax
import jax.numpy as jnp
from jax import lax
from jax.experimental import pallas as pl
from jax.experimental.pallas import tpu as pltpu
from jax.experimental.pallas import tpu_sc as plsc

NC = 2
NS = 16
NW = NC * NS

D = 64
LANES = 16
SUB = 100


def _sc_body(seq_len, nseq, nsub, n_chunks,
             idx_hbm, pos_hbm, token_hbm, out_hbm,
             idx_v, buf, pos_v, gsem):
    ch = nseq * seq_len
    wid = lax.axis_index("s") * NC + lax.axis_index("c")
    sub0 = wid * (n_chunks * nsub)

    pltpu.sync_copy(pos_hbm, pos_v)

    def chunk_body(g, carry):
        sc0 = sub0 + g * nsub
        pltpu.sync_copy(idx_hbm.at[pl.ds(sc0, nsub)], idx_v)
        copies = []
        for j in range(nsub):
            copies.append(pltpu.async_copy(
                token_hbm.at[idx_v.at[j]],
                buf.at[pl.ds(j * SUB, SUB)], gsem))
        for cp in copies:
            cp.wait()

        def add_pos(s, c2):
            for c in range(D // LANES):
                pv = pos_v[s, pl.ds(c * LANES, LANES)]
                for q in range(nseq):
                    plsc.addupdate(buf.at[q * seq_len + s,
                                          pl.ds(c * LANES, LANES)], pv)
            return c2

        lax.fori_loop(0, seq_len, add_pos, 0)
        pltpu.sync_copy(buf, out_hbm.at[pl.ds(sc0 * SUB, ch)])
        return carry

    lax.fori_loop(0, n_chunks, chunk_body, 0)


@functools.partial(jax.jit, static_argnums=(3, 4))
def _sc_embed(idx2d, pos_table, token_table, batch, seq_len):
    rows = batch * seq_len
    nseq = 4
    ch = nseq * seq_len
    nsub = ch // SUB
    n_chunks = rows // (NW * ch)

    mesh = plsc.VectorSubcoreMesh(core_axis_name="c", subcore_axis_name="s",
                                  num_cores=NC, num_subcores=NS)
    body = functools.partial(_sc_body, seq_len, nseq, nsub, n_chunks)
    out = pl.kernel(
        body,
        out_type=jax.ShapeDtypeStruct((rows, D), jnp.float32),
        mesh=mesh,
        compiler_params=pltpu.CompilerParams(use_tc_tiling_on_sc=False),
        scratch_types=[
            pltpu.VMEM((nsub, SUB), jnp.int32),
            pltpu.VMEM((ch, D), jnp.float32),
            pltpu.VMEM((seq_len, D), jnp.float32),
            pltpu.SemaphoreType.DMA,
        ],
    )(idx2d, pos_table, token_table)
    return out


def kernel(inputs, token_table, pos_table):
    batch, seq_len = inputs.shape
    rows = batch * seq_len
    assert rows % (NW * 4 * seq_len) == 0 and (4 * seq_len) % (8 * SUB) == 0
    idx2d = inputs.reshape(rows // SUB, SUB)
    out = _sc_embed(idx2d, pos_table, token_table, batch, seq_len)
    return out.reshape(batch, seq_len, token_table.shape[1])


# speedup vs baseline: 1.0798x; 1.0798x over previous
import functools

import jax
import jax.numpy as jnp
from jax import lax
from jax.experimental import pallas as pl
from jax.experimental.pallas import tpu as pltpu
from jax.experimental.pallas import tpu_sc as plsc

NC = 2
NS = 16
NW = NC * NS

D = 64
LANES = 16
SUB = 100


def _sc_body(seq_len, nseq, nsub, n_chunks,
             idx_hbm, pos_hbm, token_hbm, out_hbm,
             idx_v, buf, pos_v, gsem, osem):
    ch = nseq * seq_len
    buf_bytes_rows = ch
    wid = lax.axis_index("s") * NC + lax.axis_index("c")
    sub0 = wid * (n_chunks * nsub)

    pltpu.sync_copy(pos_hbm, pos_v)

    def fire_gathers(g_sub0, b):
        for j in range(nsub):
            pltpu.async_copy(
                token_hbm.at[idx_v[b].at[j]],
                buf[b].at[pl.ds(j * SUB, SUB)], gsem[b])

    def drain(sem, b):
        pltpu.make_async_copy(
            token_hbm.at[pl.ds(0, buf_bytes_rows)], buf[b], sem).wait()

    def add_pos(b):
        def s_body(s, c2):
            for c in range(D // LANES):
                pv = pos_v[s, pl.ds(c * LANES, LANES)]
                for q in range(nseq):
                    plsc.addupdate(buf[b].at[q * seq_len + s,
                                             pl.ds(c * LANES, LANES)], pv)
            return c2
        lax.fori_loop(0, seq_len, s_body, 0)

    pltpu.sync_copy(idx_hbm.at[pl.ds(sub0, nsub)], idx_v[0])
    fire_gathers(sub0, 0)

    def pair_body(p, carry):
        for b in range(2):
            g = 2 * p + b
            nb = 1 - b

            @pl.when(g + 1 < n_chunks)
            def _prefetch():
                pltpu.sync_copy(
                    idx_hbm.at[pl.ds(sub0 + (g + 1) * nsub, nsub)], idx_v[nb])

                @pl.when(g >= 1)
                def _wait_prev_writeout():
                    drain(osem[nb], nb)

                fire_gathers(sub0 + (g + 1) * nsub, nb)

            drain(gsem[b], b)
            add_pos(b)
            pltpu.async_copy(
                buf[b], out_hbm.at[pl.ds((sub0 + g * nsub) * SUB, ch)],
                osem[b])
        return carry

    lax.fori_loop(0, n_chunks // 2, pair_body, 0)
    drain(osem[0], 0)
    drain(osem[1], 1)


@functools.partial(jax.jit, static_argnums=(3, 4))
def _sc_embed(idx2d, pos_table, token_table, batch, seq_len):
    rows = batch * seq_len
    nseq = 4
    ch = nseq * seq_len
    nsub = ch // SUB
    n_chunks = rows // (NW * ch)

    mesh = plsc.VectorSubcoreMesh(core_axis_name="c", subcore_axis_name="s",
                                  num_cores=NC, num_subcores=NS)
    body = functools.partial(_sc_body, seq_len, nseq, nsub, n_chunks)
    out = pl.kernel(
        body,
        out_type=jax.ShapeDtypeStruct((rows, D), jnp.float32),
        mesh=mesh,
        compiler_params=pltpu.CompilerParams(use_tc_tiling_on_sc=False),
        scratch_types=[
            [pltpu.VMEM((nsub, SUB), jnp.int32)] * 2,
            [pltpu.VMEM((ch, D), jnp.float32)] * 2,
            pltpu.VMEM((seq_len, D), jnp.float32),
            [pltpu.SemaphoreType.DMA] * 2,
            [pltpu.SemaphoreType.DMA] * 2,
        ],
    )(idx2d, pos_table, token_table)
    return out


def kernel(inputs, token_table, pos_table):
    batch, seq_len = inputs.shape
    rows = batch * seq_len
    assert rows % (NW * 8 * seq_len) == 0 and (4 * seq_len) % (8 * SUB) == 0
    idx2d = inputs.reshape(rows // SUB, SUB)
    out = _sc_embed(idx2d, pos_table, token_table, batch, seq_len)
    return out.reshape(batch, seq_len, token_table.shape[1])
